# baseline (speedup 1.0000x reference)
import functools

import numpy as np
import jax
import jax.numpy as jnp
from jax import lax
from jax.experimental import pallas as pl
from jax.experimental.pallas import tpu as pltpu
from jax.experimental.pallas import tpu_sc as plsc

_B, _LQ, _E, _NH, _NL, _NP, _HD = 2, 5440, 256, 8, 4, 4, 32
_SHAPES = ((64, 64), (32, 32), (16, 16), (8, 8))
_LV = sum(h * w for h, w in _SHAPES)
_STARTS = (0, 4096, 5120, 5376)
_NQ = _B * _LQ
_NPAIR = _NH * _NL * _NP * 2
_NWGT = 2 * _NPAIR
_BLK = 1088
_GRID = _NQ // _BLK
_BPG = _GRID // _B

_NW = 32
_QW = _NQ // _NW
_C = 5
_NCH = _QW // _C

_cols = np.arange(128)
_hcol = _cols // 16
_lcol = (_cols // 4) % 4
_Wc = np.array([_SHAPES[l][1] for l in _lcol], np.float32)[None]
_Hc = np.array([_SHAPES[l][0] for l in _lcol], np.float32)[None]
_Wc_i = _Wc.astype(np.int32)
_ADDB = (_hcol * _LV + np.array([_STARTS[l] for l in _lcol])).astype(
    np.int32)[None]

_PIDX = np.zeros((_E,), np.int64)
for _h in range(_NH):
    for _i in range(16):
        _PIDX[_h * 32 + 2 * _i] = _h * 32 + _i
        _PIDX[_h * 32 + 2 * _i + 1] = _h * 32 + 16 + _i

_G = (( _cols[:, None] // 16) == (_cols[None, :] // 16)).astype(np.float32)

_SCAT = np.zeros((2 * _NL, 2 * 128), np.float32)
for _c in range(128):
    _SCAT[2 * _lcol[_c], _c] = _Wc[0, _c]
    _SCAT[2 * _lcol[_c] + 1, 128 + _c] = _Hc[0, _c]


def _tc1_body(qr_ref, refp_ref, vv_ref, vvs_ref, WvT_ref, bv_ref, WcatT_ref,
              bcat_ref, scat_ref, g_ref, cf_ref, ci_ref,
              tab_ref, idx_ref, w_ref):
    i = pl.program_id(0)
    f32 = jnp.float32
    hp = lax.Precision.HIGHEST
    Wc = cf_ref[0:1, :]
    Hc = cf_ref[1:2, :]
    Wc_i = ci_ref[0:1, :]
    addb = ci_ref[1:2, :]
    val = (jnp.dot(vv_ref[...], WvT_ref[...], preferred_element_type=f32,
                   precision=hp) + bv_ref[...])
    valn = (jnp.dot(vvs_ref[...], WvT_ref[...], preferred_element_type=f32,
                    precision=hp) + bv_ref[...])
    for h in range(_NH):
        tab_ref[h, :, 0:32] = val[:, h * 32:(h + 1) * 32].astype(jnp.bfloat16)
        tab_ref[h, :, 32:64] = valn[:, h * 32:(h + 1) * 32].astype(
            jnp.bfloat16)
    cat = (jnp.dot(qr_ref[...], WcatT_ref[...], preferred_element_type=f32,
                   precision=hp)
           + bcat_ref[...])
    offx = cat[:, 0:128]
    offy = cat[:, 128:256]
    a = cat[:, 256:384]
    m = jnp.max(a, axis=1, keepdims=True)
    e = jnp.exp(a - m)
    denom = jnp.dot(e, g_ref[...], preferred_element_type=f32, precision=hp)
    attn = e / denom
    refp = refp_ref[...]
    mask = scat_ref[...]
    rx = jnp.zeros_like(offx)
    ry = jnp.zeros_like(offx)
    for l in range(_NL):
        rx = rx + refp[:, 2 * l:2 * l + 1] * mask[2 * l:2 * l + 1, 0:128]
        ry = ry + refp[:, 2 * l + 1:2 * l + 2] * mask[2 * l + 1:2 * l + 2,
                                                      128:256]
    x = rx + offx - 0.5
    y = ry + offy - 0.5
    x0 = jnp.floor(x)
    y0 = jnp.floor(y)
    fx = x - x0
    fy = y - y0
    bb = (i // _BPG) * (_NH * _LV)

    wx0 = 1.0 - fx
    wx1 = fx
    in_mid = ((x0 >= 0.0) & (x0 <= Wc - 2.0)).astype(f32)
    s0 = wx0 * in_mid + wx1 * (x0 == -1.0).astype(f32)
    s1 = wx1 * in_mid + wx0 * (x0 == Wc - 1.0).astype(f32)
    px = jnp.clip(x0, 0.0, Wc - 2.0).astype(jnp.int32)

    idx_cols = []
    w_cols = []
    for yp in range(2):
        yv = y0 + float(yp)
        wyv = (1.0 - fy) if yp == 0 else fy
        vy = ((yv >= 0.0) & (yv <= Hc - 1.0)).astype(f32)
        yc = jnp.clip(yv, 0.0, Hc - 1.0).astype(jnp.int32)
        idx_cols.append(yc * Wc_i + px + addb + bb)
        aw = attn * wyv * vy
        w_cols.append(aw * s0)
        w_cols.append(aw * s1)
    idx_ref[...] = jnp.concatenate(idx_cols, axis=1)
    w_ref[...] = jnp.concatenate(w_cols, axis=1)


def _tc2_body(x_ref, WoT_ref, bo_ref, o_ref):
    o_ref[...] = (
        jnp.dot(x_ref[...], WoT_ref[...], preferred_element_type=jnp.float32,
                precision=lax.Precision.HIGHEST)
        + bo_ref[...])


def _splat(v, k):
    idx = jnp.full((16, 1), k, jnp.int32)
    dn = lax.GatherDimensionNumbers(
        offset_dims=(), collapsed_slice_dims=(0,), start_index_map=(0,))
    return lax.gather(v, idx, dn, (1,),
                      mode=lax.GatherScatterMode.PROMISE_IN_BOUNDS)


def _sc_body(table_hbm, idx_hbm, w_hbm, out_hbm,
             idx0, idx1, w0, w1, g0, g1, o0, o1,
             si0, si1, sg0, sg1, so0, so1):
    wid = lax.axis_index("c") * 16 + lax.axis_index("s")
    base_q = wid * _QW
    idxv = (idx0, idx1)
    wv = (w0, w1)
    gv = (g0, g1)
    ov = (o0, o1)
    si = (si0, si1)
    sg = (sg0, sg1)
    so = (so0, so1)

    def fetch(t, b):
        q0 = base_q + t * _C
        pltpu.async_copy(idx_hbm.at[pl.ds(q0 * _NPAIR, _C * _NPAIR)],
                         idxv[b], si[b])
        pltpu.async_copy(w_hbm.at[pl.ds(q0 * _NWGT, _C * _NWGT)], wv[b],
                         si[b])

    def fetch_wait(b):
        pltpu.make_async_copy(idx_hbm.at[pl.ds(0, _C * _NPAIR)], idxv[b],
                              si[b]).wait()
        pltpu.make_async_copy(w_hbm.at[pl.ds(0, _C * _NWGT)], wv[b],
                              si[b]).wait()

    def fire(b):
        pltpu.async_copy(table_hbm.at[idxv[b]], gv[b], sg[b])

    def gwait(b):
        pltpu.make_async_copy(table_hbm.at[idxv[b]], gv[b], sg[b]).wait()

    def out_wait(b):
        pltpu.make_async_copy(ov[b], out_hbm.at[pl.ds(base_q, _C)],
                              so[b]).wait()

    def compute(t, b):
        gat_v = gv[b]
        w_v = wv[b]
        out_v = ov[b]

        def qh(u, _):
            q = u // _NH
            h = u % _NH
            acc0 = jnp.zeros((16,), jnp.float32)
            acc1 = jnp.zeros((16,), jnp.float32)
            for yp in range(2):
                bg = q * _NPAIR + yp * 128 + h * 16
                bw = q * _NWGT + yp * 256 + h * 16
                w0vec = w_v[pl.ds(bw, 16)]
                w1vec = w_v[pl.ds(bw + 128, 16)]
                for kk in range(16):
                    s0 = _splat(w0vec, kk)
                    s1 = _splat(w1vec, kk)
                    e0, e1 = plsc.unpack(
                        gat_v[bg + kk, pl.ds(0, 32)],
                        format=plsc.PackFormat.INTERLEAVED,
                        preferred_element_type=jnp.float32)
                    f0, f1 = plsc.unpack(
                        gat_v[bg + kk, pl.ds(32, 32)],
                        format=plsc.PackFormat.INTERLEAVED,
                        preferred_element_type=jnp.float32)
                    acc0 = acc0 + s0 * e0 + s1 * f0
                    acc1 = acc1 + s0 * e1 + s1 * f1
            out_v[q, pl.ds(h * _HD, 16)] = acc0
            out_v[q, pl.ds(h * _HD + 16, 16)] = acc1
            return _

        lax.fori_loop(0, _C * _NH, qh, None)
        pltpu.async_copy(out_v, out_hbm.at[pl.ds(base_q + t * _C, _C)], so[b])

    fetch(0, 0)
    fetch_wait(0)
    fire(0)
    fetch(1, 1)

    def pair(u, _):
        for parity in range(2):
            t = u * 2 + parity
            b = parity

            @pl.when(t + 1 < _NCH)
            def _fire_next():
                fetch_wait(1 - b)
                fire(1 - b)

            gwait(b)

            @pl.when(t >= 2)
            def _drain_out():
                out_wait(b)

            compute(t, b)

            @pl.when(t + 2 < _NCH)
            def _prefetch():
                fetch(t + 2, b)

        return _

    lax.fori_loop(0, _NCH // 2, pair, None)
    out_wait(0)
    out_wait(1)


def _sc_gather_reduce(table, idx1, w1):
    mesh = plsc.VectorSubcoreMesh(
        core_axis_name="c", subcore_axis_name="s", num_cores=2,
        num_subcores=16)
    return pl.kernel(
        _sc_body,
        out_type=jax.ShapeDtypeStruct((_NQ, _E), jnp.float32),
        mesh=mesh,
        scratch_types=[
            pltpu.VMEM((_C * _NPAIR,), jnp.int32),
            pltpu.VMEM((_C * _NPAIR,), jnp.int32),
            pltpu.VMEM((_C * _NWGT,), jnp.float32),
            pltpu.VMEM((_C * _NWGT,), jnp.float32),
            pltpu.VMEM((_C * _NPAIR, 2 * _HD), jnp.bfloat16),
            pltpu.VMEM((_C * _NPAIR, 2 * _HD), jnp.bfloat16),
            pltpu.VMEM((_C, _E), jnp.float32),
            pltpu.VMEM((_C, _E), jnp.float32),
            pltpu.SemaphoreType.DMA,
            pltpu.SemaphoreType.DMA,
            pltpu.SemaphoreType.DMA,
            pltpu.SemaphoreType.DMA,
            pltpu.SemaphoreType.DMA,
            pltpu.SemaphoreType.DMA,
        ],
        compiler_params=pltpu.CompilerParams(use_tc_tiling_on_sc=False,
                                             needs_layout_passes=False),
    )(table, idx1, w1)


def _tc1_call(qr, refp, vv, vvs, WvT, bv2, WcatT, bcat, scat, g,
              interpret=False):
    f32 = jnp.float32
    cf = jnp.asarray(np.concatenate([_Wc, _Hc], axis=0))
    ci = jnp.asarray(np.concatenate([_Wc_i, _ADDB], axis=0))
    return pl.pallas_call(
        _tc1_body,
        grid=(_GRID,),
        in_specs=[
            pl.BlockSpec((_BLK, _E), lambda i: (i, 0)),
            pl.BlockSpec((_BLK, 2 * _NL), lambda i: (i, 0)),
            pl.BlockSpec((_BLK, _E), lambda i: (i, 0)),
            pl.BlockSpec((_BLK, _E), lambda i: (i, 0)),
            pl.BlockSpec((_E, _E), lambda i: (0, 0)),
            pl.BlockSpec((1, _E), lambda i: (0, 0)),
            pl.BlockSpec((_E, 384), lambda i: (0, 0)),
            pl.BlockSpec((1, 384), lambda i: (0, 0)),
            pl.BlockSpec((2 * _NL, 256), lambda i: (0, 0)),
            pl.BlockSpec((128, 128), lambda i: (0, 0)),
            pl.BlockSpec((2, 128), lambda i: (0, 0)),
            pl.BlockSpec((2, 128), lambda i: (0, 0)),
        ],
        out_specs=[
            pl.BlockSpec((_NH, _BLK, 2 * _HD),
                         lambda i: (i // _BPG, i % _BPG, 0)),
            pl.BlockSpec((_BLK, _NPAIR), lambda i: (i, 0)),
            pl.BlockSpec((_BLK, _NWGT), lambda i: (i, 0)),
        ],
        out_shape=[
            jax.ShapeDtypeStruct((_B * _NH, _LV, 2 * _HD), jnp.bfloat16),
            jax.ShapeDtypeStruct((_NQ, _NPAIR), jnp.int32),
            jax.ShapeDtypeStruct((_NQ, _NWGT), f32),
        ],
        interpret=interpret,
    )(qr, refp, vv, vvs, WvT, bv2, WcatT, bcat, scat, g, cf, ci)


def _tc2_call(x, WoT, bo2, interpret=False):
    return pl.pallas_call(
        _tc2_body,
        grid=(_GRID,),
        in_specs=[
            pl.BlockSpec((_BLK, _E), lambda i: (i, 0)),
            pl.BlockSpec((_E, _E), lambda i: (0, 0)),
            pl.BlockSpec((1, _E), lambda i: (0, 0)),
        ],
        out_specs=pl.BlockSpec((_BLK, _E), lambda i: (i, 0)),
        out_shape=jax.ShapeDtypeStruct((_NQ, _E), jnp.float32),
        interpret=interpret,
    )(x, WoT, bo2)


def kernel(query, reference_points, value, spatial_shapes, Wv, bv, Woff, boff,
           Wa, ba, Wo, bo):
    del spatial_shapes
    qr = query.reshape(_NQ, _E)
    refp = reference_points.reshape(_NQ, 2 * _NL)
    vv = value.reshape(_B * _LV, _E)
    vvs = jnp.concatenate([vv[1:], vv[:1]], axis=0)
    WcatT = jnp.concatenate([Woff[0::2], Woff[1::2], Wa], axis=0).T
    bcat = jnp.concatenate([boff[0::2], boff[1::2], ba])[None]
    tab, idx, w = _tc1_call(qr, refp, vv, vvs, Wv[_PIDX].T, bv[_PIDX][None],
                            WcatT, bcat, jnp.asarray(_SCAT), jnp.asarray(_G))
    rows = _sc_gather_reduce(tab.reshape(_B * _NH * _LV, 2 * _HD),
                             idx.reshape(-1), w.reshape(-1))
    out = _tc2_call(rows, Wo.T, bo[None])
    return out.reshape(_B, _LQ, _E)


# speedup vs baseline: 1.3988x; 1.3988x over previous
import functools

import numpy as np
import jax
import jax.numpy as jnp
from jax import lax
from jax.experimental import pallas as pl
from jax.experimental.pallas import tpu as pltpu
from jax.experimental.pallas import tpu_sc as plsc

_B, _LQ, _E, _NH, _NL, _NP, _HD = 2, 5440, 256, 8, 4, 4, 32
_SHAPES = ((64, 64), (32, 32), (16, 16), (8, 8))
_LV = sum(h * w for h, w in _SHAPES)
_STARTS = (0, 4096, 5120, 5376)
_NQ = _B * _LQ
_ROWS = _NH * _NL * _NP * 4
_BLK = 1088
_GRID = _NQ // _BLK
_BPG = _GRID // _B

_NW = 32
_QW = _NQ // _NW
_C = 4
_NCH = _QW // _C

_cols = np.arange(128)
_hcol = _cols // 16
_lcol = (_cols // 4) % 4
_Wc = np.array([_SHAPES[l][1] for l in _lcol], np.float32)[None]
_Hc = np.array([_SHAPES[l][0] for l in _lcol], np.float32)[None]
_Wc_i = _Wc.astype(np.int32)
_ADDB = (np.array([_STARTS[l] for l in _lcol]) * _NH + _hcol).astype(np.int32)[None]

_PIDX = np.zeros((_E,), np.int64)
for _h in range(_NH):
    for _i in range(16):
        _PIDX[_h * 32 + 2 * _i] = _h * 32 + _i
        _PIDX[_h * 32 + 2 * _i + 1] = _h * 32 + 16 + _i

_G = (( _cols[:, None] // 16) == (_cols[None, :] // 16)).astype(np.float32)

_SCAT = np.zeros((2 * _NL, 2 * 128), np.float32)
for _c in range(128):
    _SCAT[2 * _lcol[_c], _c] = _Wc[0, _c]
    _SCAT[2 * _lcol[_c] + 1, 128 + _c] = _Hc[0, _c]


def _tc1_body(qr_ref, refp_ref, vv_ref, WvT_ref, bv_ref, WcatT_ref, bcat_ref,
              scat_ref, g_ref, cf_ref, ci_ref, val_ref, idx_ref, w_ref):
    i = pl.program_id(0)
    f32 = jnp.float32
    hp = lax.Precision.HIGHEST
    Wc = cf_ref[0:1, :]
    Hc = cf_ref[1:2, :]
    Wc_i = ci_ref[0:1, :]
    addb = ci_ref[1:2, :]
    val_ref[...] = (
        jnp.dot(vv_ref[...], WvT_ref[...], preferred_element_type=f32,
                precision=hp)
        + bv_ref[...]).astype(jnp.bfloat16)
    cat = (jnp.dot(qr_ref[...], WcatT_ref[...], preferred_element_type=f32,
                   precision=hp)
           + bcat_ref[...])
    offx = cat[:, 0:128]
    offy = cat[:, 128:256]
    a = cat[:, 256:384]
    m = jnp.max(a, axis=1, keepdims=True)
    e = jnp.exp(a - m)
    denom = jnp.dot(e, g_ref[...], preferred_element_type=f32, precision=hp)
    attn = e / denom
    refp = refp_ref[...]
    mask = scat_ref[...]
    rx = jnp.zeros_like(cat[:, 0:128])
    ry = jnp.zeros_like(cat[:, 0:128])
    for l in range(_NL):
        rx = rx + refp[:, 2 * l:2 * l + 1] * mask[2 * l:2 * l + 1, 0:128]
        ry = ry + refp[:, 2 * l + 1:2 * l + 2] * mask[2 * l + 1:2 * l + 2,
                                                      128:256]
    x = rx + offx - 0.5
    y = ry + offy - 0.5
    x0 = jnp.floor(x)
    y0 = jnp.floor(y)
    fx = x - x0
    fy = y - y0

    def corner(xi, yi, wgt):
        vx = (xi >= 0.0) & (xi <= Wc - 1.0)
        vy = (yi >= 0.0) & (yi <= Hc - 1.0)
        xc = jnp.clip(xi, 0.0, Wc - 1.0).astype(jnp.int32)
        yc = jnp.clip(yi, 0.0, Hc - 1.0).astype(jnp.int32)
        row = (yc * Wc_i + xc) * _NH + addb
        wq = attn * wgt * (vx & vy).astype(f32)
        return row, wq

    r00, w00 = corner(x0, y0, (1.0 - fx) * (1.0 - fy))
    r10, w10 = corner(x0 + 1.0, y0, fx * (1.0 - fy))
    r01, w01 = corner(x0, y0 + 1.0, (1.0 - fx) * fy)
    r11, w11 = corner(x0 + 1.0, y0 + 1.0, fx * fy)
    idx_ref[...] = jnp.concatenate([r00, r10, r01, r11], axis=1)
    w_ref[...] = jnp.concatenate([w00, w10, w01, w11], axis=1)


def _tc2_body(x_ref, WoT_ref, bo_ref, o_ref):
    o_ref[...] = (
        jnp.dot(x_ref[...], WoT_ref[...], preferred_element_type=jnp.float32,
                precision=lax.Precision.HIGHEST)
        + bo_ref[...])


def _splat(v, k):
    idx = jnp.full((16, 1), k, jnp.int32)
    dn = lax.GatherDimensionNumbers(
        offset_dims=(), collapsed_slice_dims=(0,), start_index_map=(0,))
    return lax.gather(v, idx, dn, (1,),
                      mode=lax.GatherScatterMode.PROMISE_IN_BOUNDS)


def _sc_body(table_hbm, idx_hbm, w_hbm, out_hbm,
             shared_tab, idx0, idx1, w0, w1, g0, g1, o0, o1,
             si0, si1, sg0, sg1, so0, so1):
    core = lax.axis_index("c")
    sid = lax.axis_index("s")
    wid = core * 16 + sid
    base_q = wid * _QW
    tchunk = _LV * _NH // 16
    pltpu.sync_copy(
        table_hbm.at[pl.ds(core * (_LV * _NH) + sid * tchunk, tchunk)],
        shared_tab.at[pl.ds(sid * tchunk, tchunk)])
    plsc.subcore_barrier()
    idxv = (idx0, idx1)
    wv = (w0, w1)
    gv = (g0, g1)
    ov = (o0, o1)
    si = (si0, si1)
    sg = (sg0, sg1)
    so = (so0, so1)
    ng = _C * _ROWS // 128

    def fetch(t, b):
        q0 = (base_q + t * _C) * _ROWS
        pltpu.async_copy(idx_hbm.at[pl.ds(q0, _C * _ROWS)], idxv[b], si[b])
        pltpu.async_copy(w_hbm.at[pl.ds(q0, _C * _ROWS)], wv[b], si[b])

    def fetch_wait(b):
        pltpu.make_async_copy(idx_hbm.at[pl.ds(0, _C * _ROWS)], idxv[b],
                              si[b]).wait()
        pltpu.make_async_copy(w_hbm.at[pl.ds(0, _C * _ROWS)], wv[b],
                              si[b]).wait()

    def fire(b):
        pltpu.async_copy(shared_tab.at[idxv[b]], gv[b], sg[b])

    def gwait(b):
        pltpu.make_async_copy(shared_tab.at[idxv[b]], gv[b], sg[b]).wait()

    def out_wait(b):
        pltpu.make_async_copy(ov[b], out_hbm.at[pl.ds(base_q, _C)],
                              so[b]).wait()

    def compute(t, b):
        gat_v = gv[b]
        w_v = wv[b]
        out_v = ov[b]

        def qh(u, _):
            q = u // _NH
            h = u % _NH
            acc0 = jnp.zeros((16,), jnp.float32)
            acc1 = jnp.zeros((16,), jnp.float32)
            for c in range(4):
                base = q * _ROWS + c * 128 + h * 16
                wvec = w_v[pl.ds(base, 16)]
                for kk in range(16):
                    ws = _splat(wvec, kk)
                    e0, e1 = plsc.unpack(
                        gat_v[base + kk, :],
                        format=plsc.PackFormat.INTERLEAVED,
                        preferred_element_type=jnp.float32)
                    acc0 = acc0 + ws * e0
                    acc1 = acc1 + ws * e1
            out_v[q, pl.ds(h * _HD, 16)] = acc0
            out_v[q, pl.ds(h * _HD + 16, 16)] = acc1
            return _

        lax.fori_loop(0, _C * _NH, qh, None)
        pltpu.async_copy(out_v, out_hbm.at[pl.ds(base_q + t * _C, _C)], so[b])

    fetch(0, 0)
    fetch_wait(0)
    fire(0)
    fetch(1, 1)

    def pair(u, _):
        for parity in range(2):
            t = u * 2 + parity
            b = parity

            @pl.when(t + 1 < _NCH)
            def _fire_next():
                fetch_wait(1 - b)
                fire(1 - b)

            gwait(b)

            @pl.when(t >= 2)
            def _drain_out():
                out_wait(b)

            compute(t, b)

            @pl.when(t + 2 < _NCH)
            def _prefetch():
                fetch(t + 2, b)

        return _

    lax.fori_loop(0, _NCH // 2, pair, None)
    if _NCH % 2:
        gwait(0)
        out_wait(0)
        compute(_NCH - 1, 0)
    out_wait(1)
    out_wait(0)


def _sc_gather_reduce(table, idx1, w1):
    mesh = plsc.VectorSubcoreMesh(
        core_axis_name="c", subcore_axis_name="s", num_cores=2,
        num_subcores=16)
    return pl.kernel(
        _sc_body,
        out_type=jax.ShapeDtypeStruct((_NQ, _E), jnp.float32),
        mesh=mesh,
        scratch_types=[
            pltpu.VMEM_SHARED((_LV * _NH, _HD), jnp.bfloat16),
            pltpu.VMEM((_C * _ROWS,), jnp.int32),
            pltpu.VMEM((_C * _ROWS,), jnp.int32),
            pltpu.VMEM((_C * _ROWS,), jnp.float32),
            pltpu.VMEM((_C * _ROWS,), jnp.float32),
            pltpu.VMEM((_C * _ROWS, _HD), jnp.bfloat16),
            pltpu.VMEM((_C * _ROWS, _HD), jnp.bfloat16),
            pltpu.VMEM((_C, _E), jnp.float32),
            pltpu.VMEM((_C, _E), jnp.float32),
            pltpu.SemaphoreType.DMA,
            pltpu.SemaphoreType.DMA,
            pltpu.SemaphoreType.DMA,
            pltpu.SemaphoreType.DMA,
            pltpu.SemaphoreType.DMA,
            pltpu.SemaphoreType.DMA,
        ],
        compiler_params=pltpu.CompilerParams(use_tc_tiling_on_sc=False,
                                             needs_layout_passes=False),
    )(table, idx1, w1)


def _tc1_call(qr, refp, vv, WvT, bv2, WcatT, bcat, scat, g, interpret=False):
    f32 = jnp.float32
    cf = jnp.asarray(np.concatenate([_Wc, _Hc], axis=0))
    ci = jnp.asarray(np.concatenate([_Wc_i, _ADDB], axis=0))
    return pl.pallas_call(
        _tc1_body,
        grid=(_GRID,),
        in_specs=[
            pl.BlockSpec((_BLK, _E), lambda i: (i, 0)),
            pl.BlockSpec((_BLK, 2 * _NL), lambda i: (i, 0)),
            pl.BlockSpec((_BLK, _E), lambda i: (i, 0)),
            pl.BlockSpec((_E, _E), lambda i: (0, 0)),
            pl.BlockSpec((1, _E), lambda i: (0, 0)),
            pl.BlockSpec((_E, 384), lambda i: (0, 0)),
            pl.BlockSpec((1, 384), lambda i: (0, 0)),
            pl.BlockSpec((2 * _NL, 256), lambda i: (0, 0)),
            pl.BlockSpec((128, 128), lambda i: (0, 0)),
            pl.BlockSpec((2, 128), lambda i: (0, 0)),
            pl.BlockSpec((2, 128), lambda i: (0, 0)),
        ],
        out_specs=[
            pl.BlockSpec((_BLK, _E), lambda i: (i, 0)),
            pl.BlockSpec((_BLK, _ROWS), lambda i: (i, 0)),
            pl.BlockSpec((_BLK, _ROWS), lambda i: (i, 0)),
        ],
        out_shape=[
            jax.ShapeDtypeStruct((_B * _LV, _E), jnp.bfloat16),
            jax.ShapeDtypeStruct((_NQ, _ROWS), jnp.int32),
            jax.ShapeDtypeStruct((_NQ, _ROWS), f32),
        ],
        interpret=interpret,
    )(qr, refp, vv, WvT, bv2, WcatT, bcat, scat, g, cf, ci)


def _tc2_call(x, WoT, bo2, interpret=False):
    return pl.pallas_call(
        _tc2_body,
        grid=(_GRID,),
        in_specs=[
            pl.BlockSpec((_BLK, _E), lambda i: (i, 0)),
            pl.BlockSpec((_E, _E), lambda i: (0, 0)),
            pl.BlockSpec((1, _E), lambda i: (0, 0)),
        ],
        out_specs=pl.BlockSpec((_BLK, _E), lambda i: (i, 0)),
        out_shape=jax.ShapeDtypeStruct((_NQ, _E), jnp.float32),
        interpret=interpret,
    )(x, WoT, bo2)


def kernel(query, reference_points, value, spatial_shapes, Wv, bv, Woff, boff,
           Wa, ba, Wo, bo):
    del spatial_shapes
    qr = query.reshape(_NQ, _E)
    refp = reference_points.reshape(_NQ, 2 * _NL)
    vv = value.reshape(_B * _LV, _E)
    WcatT = jnp.concatenate([Woff[0::2], Woff[1::2], Wa], axis=0).T
    bcat = jnp.concatenate([boff[0::2], boff[1::2], ba])[None]
    val, idx, w = _tc1_call(qr, refp, vv, Wv[_PIDX].T, bv[_PIDX][None],
                            WcatT, bcat, jnp.asarray(_SCAT), jnp.asarray(_G))
    rows = _sc_gather_reduce(val.reshape(_B * _LV * _NH, _HD),
                             idx.reshape(-1), w.reshape(-1))
    out = _tc2_call(rows, Wo.T, bo[None])
    return out.reshape(_B, _LQ, _E)
